# baseline (speedup 1.0000x reference)
import dataclasses
import functools

import jax
import jax.numpy as jnp
from jax import lax
from jax.experimental import pallas as pl
from jax.experimental.pallas import tpu as pltpu
from jax.experimental.pallas import tpu_sc as plsc

NUM_SC = 2
NUM_SUBCORES = 16
NUM_WORKERS = NUM_SC * NUM_SUBCORES
CHUNK = 128


def _sc_gather(flat_table, xcat1d, n_chunks_per_worker, f, v, d):
    total_rows = xcat1d.shape[0]
    cpw = n_chunks_per_worker
    ipw = cpw * CHUNK
    win = 8 * f
    n_win = ipw // win
    nbuf = 4 if cpw % 4 == 0 else 2
    assert cpw % nbuf == 0 and win % 16 == 0 and ipw % 8 == 0
    mesh = plsc.VectorSubcoreMesh(core_axis_name="c", subcore_axis_name="s")
    cp = pltpu.CompilerParams()
    if "needs_layout_passes" in pltpu.CompilerParams.__dataclass_fields__:
        cp = dataclasses.replace(cp, needs_layout_passes=False)

    @functools.partial(
        pl.kernel,
        out_type=jax.ShapeDtypeStruct((total_rows, d), jnp.float32),
        mesh=mesh,
        compiler_params=cp,
        scratch_types=[
            pltpu.VMEM((ipw,), jnp.int32),
            pltpu.VMEM((ipw,), jnp.int32),
        ]
        + [pltpu.VMEM((CHUNK, d), jnp.float32)] * nbuf
        + [pltpu.SemaphoreType.DMA] * (2 * nbuf),
    )
    def gather_kernel(tbl_hbm, idx_hbm, out_hbm, idx_n, idx_p, *bufs_sems):
        rbufs = bufs_sems[:nbuf]
        gsems = bufs_sems[nbuf : 2 * nbuf]
        wsems = bufs_sems[2 * nbuf :]
        wid = lax.axis_index("s") * NUM_SC + lax.axis_index("c")
        row_base = wid * ipw

        pltpu.sync_copy(idx_hbm.at[pl.ds(row_base, ipw)], idx_n)

        lanes = lax.iota(jnp.int32, 16)

        @pl.loop(0, n_win)
        def _(w):
            wbase = w * win
            for k in range(win // 16):
                j = k * 16 + lanes
                s = j & 7
                fld = j >> 3
                p = wbase + s * f + fld
                vals = plsc.load_gather(idx_n, [p])
                idx_p[pl.ds(wbase + k * 16, 16)] = vals + fld * v

        @pl.loop(0, cpw, step=nbuf)
        def _(c):
            for i in range(nbuf):

                @pl.when(c > 0)
                def _(i=i):
                    pltpu.make_async_copy(
                        rbufs[i], out_hbm.at[pl.ds(row_base, CHUNK)], wsems[i]
                    ).wait()

                pltpu.make_async_copy(
                    tbl_hbm.at[idx_p.at[pl.ds((c + i) * CHUNK, CHUNK)]],
                    rbufs[i],
                    gsems[i],
                ).start()

            for i in range(nbuf):
                pltpu.make_async_copy(
                    tbl_hbm.at[idx_p.at[pl.ds((c + i) * CHUNK, CHUNK)]],
                    rbufs[i],
                    gsems[i],
                ).wait()
                pltpu.make_async_copy(
                    rbufs[i],
                    out_hbm.at[pl.ds(row_base + (c + i) * CHUNK, CHUNK)],
                    wsems[i],
                ).start()

        for i in range(nbuf):
            pltpu.make_async_copy(
                rbufs[i], out_hbm.at[pl.ds(row_base, CHUNK)], wsems[i]
            ).wait()

    return gather_kernel(flat_table, xcat1d)


def _tc_assemble_t(emb1d, x_cont_t, cont_min2d, cont_max2d, f, d, b_total,
                   col0, prev):
    nc, bh = x_cont_t.shape
    n_emb = f * d
    n_out = n_emb + nc
    bblk = 512
    cblk0 = col0 // bblk

    def body(*refs):
        emb_ref, xc_ref, mn_ref, mx_ref = refs[-5:-1]
        o_ref = refs[-1]
        for fld in range(f):
            tile = jnp.concatenate(
                [
                    emb_ref[pl.ds((rb * f + fld) * 8, 8), :]
                    for rb in range(bblk // 8)
                ],
                axis=0,
            )
            o_ref[pl.ds(fld * d, d), :] = tile.T
        mn = mn_ref[...]
        mx = mx_ref[...]
        o_ref[pl.ds(n_emb, nc), :] = (xc_ref[...] - mn) / (mx - mn)

    data_specs = [
        pl.BlockSpec((bblk * f, d), lambda i: (i, 0)),
        pl.BlockSpec((nc, bblk), lambda i: (0, i)),
        pl.BlockSpec((nc, 1), lambda i: (0, 0)),
        pl.BlockSpec((nc, 1), lambda i: (0, 0)),
    ]
    if prev is None:
        in_specs = data_specs
        args = (emb1d, x_cont_t, cont_min2d, cont_max2d)
        aliases = {}
    else:
        in_specs = [pl.BlockSpec(memory_space=pl.ANY)] + data_specs
        args = (prev, emb1d, x_cont_t, cont_min2d, cont_max2d)
        aliases = {0: 0}

    return pl.pallas_call(
        body,
        out_shape=jax.ShapeDtypeStruct((n_out, b_total), jnp.float32),
        grid=(bh // bblk,),
        in_specs=in_specs,
        out_specs=pl.BlockSpec((n_out, bblk), lambda i: (0, cblk0 + i)),
        input_output_aliases=aliases,
    )(*args)


def kernel(x_cat, x_cont, tables, cont_min, cont_max):
    b, f = x_cat.shape
    f_, v, d = tables.shape
    flat_table = tables.reshape(f_ * v, d)
    n_slices = 4
    bh = b // n_slices

    x_cont_t = x_cont.T
    mn2 = cont_min.reshape(-1, 1)
    mx2 = cont_max.reshape(-1, 1)

    cpw = bh * f // CHUNK // NUM_WORKERS
    embs = []
    for h in range(n_slices):
        xcat1d = x_cat[h * bh : (h + 1) * bh].reshape(bh * f)
        embs.append(_sc_gather(flat_table, xcat1d, cpw, f, v, d))

    out_t = None
    for h in range(n_slices):
        out_t = _tc_assemble_t(
            embs[h],
            x_cont_t[:, h * bh : (h + 1) * bh],
            mn2,
            mx2,
            f,
            d,
            b,
            h * bh,
            out_t,
        )
    return out_t.T


# speedup vs baseline: 1.0089x; 1.0089x over previous
import dataclasses
import functools

import jax
import jax.numpy as jnp
from jax import lax
from jax.experimental import pallas as pl
from jax.experimental.pallas import tpu as pltpu
from jax.experimental.pallas import tpu_sc as plsc

NUM_SC = 2
NUM_SUBCORES = 16
NUM_WORKERS = NUM_SC * NUM_SUBCORES
CHUNK = 128


def _sc_gather(flat_table, xcat1d, n_chunks_per_worker, f, v, d):
    total_rows = xcat1d.shape[0]
    cpw = n_chunks_per_worker
    ipw = cpw * CHUNK
    win = 8 * f
    n_win = ipw // win
    nbuf = 4 if cpw % 4 == 0 else 2
    assert cpw % nbuf == 0 and win % 16 == 0 and ipw % 8 == 0
    mesh = plsc.VectorSubcoreMesh(core_axis_name="c", subcore_axis_name="s")
    cp = pltpu.CompilerParams()
    if "needs_layout_passes" in pltpu.CompilerParams.__dataclass_fields__:
        cp = dataclasses.replace(cp, needs_layout_passes=False)

    @functools.partial(
        pl.kernel,
        out_type=jax.ShapeDtypeStruct((total_rows, d), jnp.float32),
        mesh=mesh,
        compiler_params=cp,
        scratch_types=[
            pltpu.VMEM((ipw,), jnp.int32),
            pltpu.VMEM((ipw,), jnp.int32),
        ]
        + [pltpu.VMEM((CHUNK, d), jnp.float32)] * nbuf
        + [pltpu.SemaphoreType.DMA] * (2 * nbuf),
    )
    def gather_kernel(tbl_hbm, idx_hbm, out_hbm, idx_n, idx_p, *bufs_sems):
        rbufs = bufs_sems[:nbuf]
        gsems = bufs_sems[nbuf : 2 * nbuf]
        wsems = bufs_sems[2 * nbuf :]
        wid = lax.axis_index("s") * NUM_SC + lax.axis_index("c")
        row_base = wid * ipw

        pltpu.sync_copy(idx_hbm.at[pl.ds(row_base, ipw)], idx_n)

        lanes = lax.iota(jnp.int32, 16)

        @pl.loop(0, n_win)
        def _(w):
            wbase = w * win
            for k in range(win // 16):
                j = k * 16 + lanes
                s = j & 7
                fld = j >> 3
                p = wbase + s * f + fld
                vals = plsc.load_gather(idx_n, [p])
                idx_p[pl.ds(wbase + k * 16, 16)] = vals + fld * v

        @pl.loop(0, cpw, step=nbuf)
        def _(c):
            for i in range(nbuf):

                @pl.when(c > 0)
                def _(i=i):
                    pltpu.make_async_copy(
                        rbufs[i], out_hbm.at[pl.ds(row_base, CHUNK)], wsems[i]
                    ).wait()

                pltpu.make_async_copy(
                    tbl_hbm.at[idx_p.at[pl.ds((c + i) * CHUNK, CHUNK)]],
                    rbufs[i],
                    gsems[i],
                ).start()

            for i in range(nbuf):
                pltpu.make_async_copy(
                    tbl_hbm.at[idx_p.at[pl.ds((c + i) * CHUNK, CHUNK)]],
                    rbufs[i],
                    gsems[i],
                ).wait()
                pltpu.make_async_copy(
                    rbufs[i],
                    out_hbm.at[pl.ds(row_base + (c + i) * CHUNK, CHUNK)],
                    wsems[i],
                ).start()

        for i in range(nbuf):
            pltpu.make_async_copy(
                rbufs[i], out_hbm.at[pl.ds(row_base, CHUNK)], wsems[i]
            ).wait()

    return gather_kernel(flat_table, xcat1d)


def _tc_assemble_t(emb1d, x_cont_t, cont_min2d, cont_max2d, f, d, b_total,
                   col0, prev):
    nc, bh = x_cont_t.shape
    n_emb = f * d
    n_out = n_emb + nc
    bblk = 512
    cblk0 = col0 // bblk

    def body(*refs):
        emb_ref, xc_ref, mn_ref, mx_ref = refs[-5:-1]
        o_ref = refs[-1]
        for fld in range(f):
            tile = jnp.concatenate(
                [
                    emb_ref[pl.ds((rb * f + fld) * 8, 8), :]
                    for rb in range(bblk // 8)
                ],
                axis=0,
            )
            o_ref[pl.ds(fld * d, d), :] = tile.T
        mn = mn_ref[...]
        mx = mx_ref[...]
        o_ref[pl.ds(n_emb, nc), :] = (xc_ref[...] - mn) / (mx - mn)

    data_specs = [
        pl.BlockSpec((bblk * f, d), lambda i: (i, 0)),
        pl.BlockSpec((nc, bblk), lambda i: (0, i)),
        pl.BlockSpec((nc, 1), lambda i: (0, 0)),
        pl.BlockSpec((nc, 1), lambda i: (0, 0)),
    ]
    if prev is None:
        in_specs = data_specs
        args = (emb1d, x_cont_t, cont_min2d, cont_max2d)
        aliases = {}
    else:
        in_specs = [pl.BlockSpec(memory_space=pl.ANY)] + data_specs
        args = (prev, emb1d, x_cont_t, cont_min2d, cont_max2d)
        aliases = {0: 0}

    return pl.pallas_call(
        body,
        out_shape=jax.ShapeDtypeStruct((n_out, b_total), jnp.float32),
        grid=(bh // bblk,),
        in_specs=in_specs,
        out_specs=pl.BlockSpec((n_out, bblk), lambda i: (0, cblk0 + i)),
        input_output_aliases=aliases,
    )(*args)


def kernel(x_cat, x_cont, tables, cont_min, cont_max):
    b, f = x_cat.shape
    f_, v, d = tables.shape
    flat_table = tables.reshape(f_ * v, d)
    n_slices = 2
    bh = b // n_slices

    x_cont_t = x_cont.T
    mn2 = cont_min.reshape(-1, 1)
    mx2 = cont_max.reshape(-1, 1)

    cpw = bh * f // CHUNK // NUM_WORKERS
    embs = []
    for h in range(n_slices):
        xcat1d = x_cat[h * bh : (h + 1) * bh].reshape(bh * f)
        embs.append(_sc_gather(flat_table, xcat1d, cpw, f, v, d))

    out_t = None
    for h in range(n_slices):
        out_t = _tc_assemble_t(
            embs[h],
            x_cont_t[:, h * bh : (h + 1) * bh],
            mn2,
            mx2,
            f,
            d,
            b,
            h * bh,
            out_t,
        )
    return out_t.T
